# Initial kernel scaffold; baseline (speedup 1.0000x reference)
import jax
import jax.numpy as jnp
from jax.experimental import pallas as pl


def kernel(durations, phoneme, frame):
    raise NotImplementedError("write your pallas kernel here")


# speedup vs baseline: 1.6915x; 1.6915x over previous
import jax
import jax.numpy as jnp
from jax.experimental import pallas as pl


def _cumsum_kernel(d_ref, o_ref):
    x = d_ref[...]
    rows, n = x.shape
    shift = 1
    while shift < n:
        zeros = jnp.zeros((rows, shift), dtype=x.dtype)
        x = x + jnp.concatenate([zeros, x[:, : n - shift]], axis=1)
        shift *= 2
    o_ref[...] = x


def kernel(durations, phoneme, frame):
    del phoneme, frame
    return pl.pallas_call(
        _cumsum_kernel,
        out_shape=jax.ShapeDtypeStruct(durations.shape, durations.dtype),
    )(durations)
